# baseline (speedup 1.0000x reference)
import functools

import jax
import jax.numpy as jnp
from jax import lax
from jax.experimental import pallas as pl
from jax.experimental.pallas import tpu as pltpu
from jax.experimental.pallas import tpu_sc as plsc

N_NODES = 10000
D = 128
H = 256
E_EDGES = 320000
NEXP = 4

NW = 32
ROWS_TOTAL = 2 * E_EDGES
RPW = ROWS_TOTAL // NW
CHUNK = 80
NCHUNK = RPW // CHUNK
KBUF = 5

BE = 2560
NB = E_EDGES // BE
WCOL = 640


def _sc_gather(z, idx3):
    mesh = plsc.VectorSubcoreMesh(core_axis_name="c", subcore_axis_name="s")

    @functools.partial(
        pl.kernel,
        mesh=mesh,
        out_type=jax.ShapeDtypeStruct((ROWS_TOTAL, D), jnp.float32),
        scratch_types=(
            [pltpu.VMEM((NCHUNK, CHUNK), jnp.int32)]
            + [pltpu.VMEM((CHUNK, D), jnp.float32) for _ in range(KBUF)]
            + [pltpu.SemaphoreType.DMA for _ in range(2 * KBUF)]
        ),
    )
    def gather_kernel(z_hbm, idx_hbm, out_hbm, idx_v, *rest):
        rows = rest[:KBUF]
        gsem = rest[KBUF:2 * KBUF]
        osem = rest[2 * KBUF:]
        wid = lax.axis_index("s") * 2 + lax.axis_index("c")
        base = wid * RPW
        pltpu.sync_copy(idx_hbm.at[wid], idx_v)

        def outer(j, _):
            c0 = j * KBUF
            handles = []
            for s in range(KBUF):
                @pl.when(j > 0)
                def _wait_out(s=s):
                    pltpu.make_async_copy(
                        rows[s], out_hbm.at[pl.ds(0, CHUNK)], osem[s]
                    ).wait()
                handles.append(
                    pltpu.async_copy(z_hbm.at[idx_v.at[c0 + s]], rows[s], gsem[s])
                )
            for s in range(KBUF):
                handles[s].wait()
                pltpu.async_copy(
                    rows[s],
                    out_hbm.at[pl.ds(base + (c0 + s) * CHUNK, CHUNK)],
                    osem[s],
                )
            return ()

        lax.fori_loop(0, NCHUNK // KBUF, outer, (), unroll=False)
        for s in range(KBUF):
            pltpu.make_async_copy(
                rows[s], out_hbm.at[pl.ds(0, CHUNK)], osem[s]
            ).wait()

    return gather_kernel(z, idx3)


def _tc_body(zu_ref, zv_ref, w1a_ref, w23_ref, w4_ref, g_ref, b1a_ref,
             b23_ref, b4_ref, gb_ref, w2_ref, b2_ref,
             out_ref, aux_ref, acc_ref):
    i = pl.program_id(0)
    zu = zu_ref[...]
    zv = zv_ref[...]
    dd = jnp.abs(zu - zv)
    mm = zu * zv
    X = jnp.concatenate([zu, zv, dd, mm], axis=1)
    Xb = X.astype(jnp.bfloat16)
    h1 = jnp.maximum(
        jnp.dot(Xb[:, 0:2 * D], w1a_ref[...],
                preferred_element_type=jnp.float32) + b1a_ref[...], 0.0)
    h23 = jnp.maximum(
        jnp.dot(Xb[:, 2 * D:4 * D], w23_ref[...],
                preferred_element_type=jnp.float32) + b23_ref[...], 0.0)
    h4 = jnp.maximum(
        jnp.dot(Xb, w4_ref[...],
                preferred_element_type=jnp.float32) + b4_ref[...], 0.0)
    gl = jnp.dot(X, g_ref[...], preferred_element_type=jnp.float32) + gb_ref[...]
    gmax = jnp.max(gl, axis=1, keepdims=True)
    ge = jnp.exp(gl - gmax)
    gsum = jnp.sum(ge, axis=1, keepdims=True)
    pmax = 1.0 / gsum
    w2 = w2_ref[...]
    b2 = b2_ref[...]
    s1 = jnp.sum(h1 * w2[0], axis=1, keepdims=True) + b2[0:1]
    s2 = jnp.sum(h23[:, 0:H] * w2[1], axis=1, keepdims=True) + b2[1:2]
    s3 = jnp.sum(h23[:, H:2 * H] * w2[2], axis=1, keepdims=True) + b2[2:3]
    s4 = jnp.sum(h4 * w2[3], axis=1, keepdims=True) + b2[3:4]
    c0 = gl[:, 0:1] >= gmax
    c1 = gl[:, 1:2] >= gmax
    c2 = gl[:, 2:3] >= gmax
    sel = jnp.where(c0, s1, jnp.where(c1, s2, jnp.where(c2, s3, s4)))
    out_ref[...] = pmax * sel

    @pl.when(i == 0)
    def _init():
        acc_ref[...] = jnp.zeros_like(acc_ref)

    acc_ref[...] += jnp.sum(ge * pmax, axis=0, keepdims=True)

    @pl.when(i == NB - 1)
    def _finish():
        avg = acc_ref[...] / float(E_EDGES)
        aux_ref[...] = (jnp.sum(avg * avg) * float(NEXP)).reshape(1, 1)


def _tc_moe(gathered, W1a, W23, W4, Gp, b1a, b23, b4, gbp, W2p, b2p):
    full = lambda shape: pl.BlockSpec(shape, lambda i: tuple(0 for _ in shape))
    return pl.pallas_call(
        _tc_body,
        grid=(NB,),
        in_specs=[
            pl.BlockSpec((BE, D), lambda i: (i, 0)),
            pl.BlockSpec((BE, D), lambda i: (i + NB, 0)),
            full((2 * D, H)),
            full((2 * D, 2 * H)),
            full((4 * D, H)),
            full((4 * D, 128)),
            full((1, H)),
            full((1, 2 * H)),
            full((1, H)),
            full((1, 128)),
            full((NEXP, H)),
            full((NEXP, 1)),
        ],
        out_specs=[
            pl.BlockSpec((BE, 1), lambda i: (i, 0)),
            pl.BlockSpec((1, 1), lambda i: (0, 0)),
        ],
        out_shape=[
            jax.ShapeDtypeStruct((E_EDGES, 1), jnp.float32),
            jax.ShapeDtypeStruct((1, 1), jnp.float32),
        ],
        scratch_shapes=[pltpu.VMEM((1, 128), jnp.float32)],
    )(gathered, gathered, W1a, W23, W4, Gp, b1a, b23, b4, gbp, W2p, b2p)


def kernel(g, z, uv, gate_W, gate_b, ec_W1, ec_b1, ec_W2, ec_b2,
           ed_W1, ed_b1, ed_W2, ed_b2, em_W1, em_b1, em_W2, em_b2,
           ea_W1, ea_b1, ea_W2, ea_b2):
    idx3 = uv.reshape(NW, NCHUNK, CHUNK)
    gathered = _sc_gather(z, idx3)

    W1a = ec_W1.astype(jnp.bfloat16)
    zblk = jnp.zeros((D, H), jnp.float32)
    W23 = jnp.concatenate([
        jnp.concatenate([ed_W1, zblk], axis=1),
        jnp.concatenate([zblk, em_W1], axis=1),
    ], axis=0).astype(jnp.bfloat16)
    W4 = ea_W1.astype(jnp.bfloat16)
    Gp = jnp.concatenate(
        [gate_W, jnp.zeros((4 * D, 128 - NEXP), jnp.float32)], axis=1)
    b1a = ec_b1[None, :]
    b23 = jnp.concatenate([ed_b1, em_b1])[None, :]
    b4 = ea_b1[None, :]
    gbp = jnp.concatenate(
        [gate_b, jnp.full((128 - NEXP,), -1e30, jnp.float32)])[None, :]
    W2p = jnp.stack([ec_W2[:, 0], ed_W2[:, 0], em_W2[:, 0], ea_W2[:, 0]])
    b2p = jnp.stack([ec_b2, ed_b2, em_b2, ea_b2])

    out, aux = _tc_moe(gathered, W1a, W23, W4, Gp, b1a, b23, b4, gbp, W2p, b2p)
    return out, aux[0, 0]
